# Initial kernel scaffold; baseline (speedup 1.0000x reference)
import jax
import jax.numpy as jnp
from jax.experimental import pallas as pl


def kernel(pred, target):
    raise NotImplementedError("write your pallas kernel here")


# speedup vs baseline: 7.0743x; 7.0743x over previous
import jax
import jax.numpy as jnp
from jax.experimental import pallas as pl
from jax.experimental.pallas import tpu as pltpu

VOCAB = 32000
PAD = 0
SMOOTH = 0.1
CONF = 1.0 - SMOOTH
EPS = SMOOTH / (VOCAB - 2)

ROWS = 256
VB = 6400


def _body(tgt_ref, x_ref, out_ref, m_ref, s_ref, sum_ref, tv_ref, p0_ref,
          acc_ref):
    i = pl.program_id(0)
    j = pl.program_id(1)
    ni = pl.num_programs(0)
    nj = pl.num_programs(1)
    x = x_ref[...]

    @pl.when(j == 0)
    def _init():
        m_ref[...] = jnp.full((ROWS, 1), -jnp.inf, jnp.float32)
        s_ref[...] = jnp.zeros((ROWS, 1), jnp.float32)
        sum_ref[...] = jnp.zeros((ROWS, 1), jnp.float32)
        tv_ref[...] = jnp.zeros((ROWS, 1), jnp.float32)
        p0_ref[...] = x[:, 0:1]

    @pl.when((i == 0) & (j == 0))
    def _init_acc():
        acc_ref[0] = 0.0

    m_old = m_ref[...]
    m_new = jnp.maximum(m_old, jnp.max(x, axis=1, keepdims=True))
    s_ref[...] = (s_ref[...] * jnp.exp(m_old - m_new)
                  + jnp.sum(jnp.exp(x - m_new), axis=1, keepdims=True))
    m_ref[...] = m_new
    sum_ref[...] += jnp.sum(x, axis=1, keepdims=True)

    tgt = tgt_ref[...]
    col = j * VB + jax.lax.broadcasted_iota(jnp.int32, (ROWS, VB), 1)
    hit = col == tgt
    tv_ref[...] += jnp.sum(jnp.where(hit, x, 0.0), axis=1, keepdims=True)

    @pl.when(j == nj - 1)
    def _fin():
        lse = m_ref[...] + jnp.log(s_ref[...])
        loss = (lse - EPS * (sum_ref[...] - p0_ref[...])
                - (CONF - EPS) * tv_ref[...])
        loss = jnp.where(tgt != PAD, loss, 0.0)
        acc_ref[0] += jnp.sum(loss)

    @pl.when((i == ni - 1) & (j == nj - 1))
    def _out():
        out_ref[0, 0] = acc_ref[0]


def kernel(pred, target):
    n = pred.shape[0] * pred.shape[1]
    pred2 = pred.reshape(n, VOCAB)
    ni = n // ROWS
    tgt = target.astype(jnp.int32).reshape(n, 1)

    out = pl.pallas_call(
        _body,
        grid=(ni, VOCAB // VB),
        in_specs=[
            pl.BlockSpec((ROWS, 1), lambda i, j: (i, 0)),
            pl.BlockSpec((ROWS, VB), lambda i, j: (i, j)),
        ],
        out_specs=pl.BlockSpec((1, 1), lambda i, j: (0, 0),
                               memory_space=pltpu.SMEM),
        out_shape=jax.ShapeDtypeStruct((1, 1), jnp.float32),
        scratch_shapes=[
            pltpu.VMEM((ROWS, 1), jnp.float32),
            pltpu.VMEM((ROWS, 1), jnp.float32),
            pltpu.VMEM((ROWS, 1), jnp.float32),
            pltpu.VMEM((ROWS, 1), jnp.float32),
            pltpu.VMEM((ROWS, 1), jnp.float32),
            pltpu.SMEM((1,), jnp.float32),
        ],
        compiler_params=pltpu.CompilerParams(
            dimension_semantics=("arbitrary", "arbitrary")),
    )(tgt, pred2)
    return out[0, 0] / n
